# baseline (speedup 1.0000x reference)
import functools

import jax
import jax.numpy as jnp
from jax import lax
from jax.experimental import pallas as pl
from jax.experimental.pallas import tpu as pltpu
from jax.experimental.pallas import tpu_sc as plsc

NC = 2
NS = 16
NW = NC * NS
LANES = 16

NBEV = 65536
C = 80
NR = 128
RROWS = NBEV // NR
RPW = NR // NW
CH = 256
IDXC = 128
NBUF = 3
P_PAD = CH
BND_PAD = 152

_GDN = lax.GatherDimensionNumbers(
    offset_dims=(), collapsed_slice_dims=(0,), start_index_map=(0,))


def _bev_body(depth_hbm, feat_hbm, idx_hbm, bnd_hbm, out_hbm,
              acc, stg0, stg1, stg2, dv0, dv1, dv2, fv0, fv1, fv2,
              bndv, sems0, sems1, sems2, semg0, semg1, semg2):
    cid = lax.axis_index("c")
    sid = lax.axis_index("s")
    wid = sid * NC + cid
    stg = (stg0, stg1, stg2)
    dv = (dv0, dv1, dv2)
    fv = (fv0, fv1, fv2)
    sem_s = (sems0, sems1, sems2)
    sem_g = (semg0, semg1, semg2)

    pltpu.sync_copy(bnd_hbm, bndv)
    resid = bndv[pl.ds(NR + 1, LANES)][0].astype(jnp.float32)
    rsplat = jnp.full((LANES,), 1.0, dtype=jnp.float32) * resid

    for r in range(RPW):
        rng = wid * RPW + r
        base = rng * RROWS

        def zbody(i, carry):
            acc[pl.ds(i * LANES, LANES)] = rsplat
            return carry
        lax.fori_loop(0, RROWS * C // LANES, zbody, 0)

        lo = bndv[pl.ds(rng, LANES)][0]
        hi = bndv[pl.ds(rng + 1, LANES)][0]
        lo_a = (lo // 8) * 8
        n = hi - lo_a
        nch = (n + CH - 1) // CH

        def fire_stage(j, k):
            offs = lo_a + j * CH
            pltpu.async_copy(idx_hbm.at[:, pl.ds(offs, CH)], stg[k], sem_s[k])

        def wait_stage(j, k):
            pltpu.make_async_copy(idx_hbm.at[:, pl.ds(lo_a + j * CH, CH)],
                                  stg[k], sem_s[k]).wait()

        def fire_gathers(k):
            for h in range(CH // IDXC):
                sl = pl.ds(h * IDXC, IDXC)
                pltpu.async_copy(depth_hbm.at[stg[k].at[1, sl]],
                                 dv[k].at[sl], sem_g[k])
                pltpu.async_copy(feat_hbm.at[stg[k].at[2, sl]],
                                 fv[k].at[sl, :], sem_g[k])

        def wait_gathers(k):
            for h in range(CH // IDXC):
                sl = pl.ds(h * IDXC, IDXC)
                pltpu.make_async_copy(depth_hbm.at[stg[k].at[1, sl]],
                                      dv[k].at[sl], sem_g[k]).wait()
                pltpu.make_async_copy(feat_hbm.at[stg[k].at[2, sl]],
                                      fv[k].at[sl, :], sem_g[k]).wait()

        def compute(k):
            iota = lax.iota(jnp.int32, LANES)

            def pt_body(p):
                gp = (p // LANES) * LANES
                lane = p - gp
                rb16 = stg[k][0, pl.ds(gp, LANES)]
                d16 = dv[k][pl.ds(gp, LANES)]
                lanev = jnp.full((LANES, 1), lane, dtype=jnp.int32)
                bevb = lax.gather(rb16, lanev, _GDN, (1,),
                                  mode=lax.GatherScatterMode.PROMISE_IN_BOUNDS)
                db = lax.gather(d16, lanev, _GDN, (1,),
                                mode=lax.GatherScatterMode.PROMISE_IN_BOUNDS)
                okv = jnp.logical_and(bevb >= base, bevb < base + RROWS)
                idx0 = (bevb - base) * C + iota
                for cg in range(C // LANES):
                    x = fv[k][p, pl.ds(cg * LANES, LANES)]
                    plsc.addupdate_scatter(
                        acc, [idx0 + (cg * LANES) if cg else idx0],
                        x * db, mask=okv)
            plsc.parallel_loop(0, CH, 1, unroll=12)(pt_body)

        @pl.when(nch > 0)
        def _():
            fire_stage(0, 0)
            wait_stage(0, 0)
            fire_gathers(0)

        @pl.when(nch > 1)
        def _():
            fire_stage(1, 1)

        def ring_body(jj, carry):
            for b in range(NBUF):
                j = jj * NBUF + b

                @pl.when(j < nch)
                def _():
                    @pl.when(j + 2 < nch)
                    def _():
                        fire_stage(j + 2, (b + 2) % NBUF)

                    @pl.when(j + 1 < nch)
                    def _():
                        wait_stage(j + 1, (b + 1) % NBUF)
                        fire_gathers((b + 1) % NBUF)

                    wait_gathers(b)
                    compute(b)
            return carry
        lax.fori_loop(0, (nch + NBUF - 1) // NBUF, ring_body, 0)

        pltpu.sync_copy(acc, out_hbm.at[pl.ds(base * C, RROWS * C)])


@functools.partial(jax.jit, donate_argnums=())
def _bev_pool(depth_flat, feat2, idx3, bnd):
    mesh = plsc.VectorSubcoreMesh(core_axis_name="c", subcore_axis_name="s",
                                  num_cores=NC, num_subcores=NS)
    f = pl.kernel(
        _bev_body,
        out_type=jax.ShapeDtypeStruct((NBEV * C,), jnp.float32),
        mesh=mesh,
        scratch_types=[
            pltpu.VMEM((RROWS * C,), jnp.float32),
            pltpu.VMEM((3, CH), jnp.int32),
            pltpu.VMEM((3, CH), jnp.int32),
            pltpu.VMEM((3, CH), jnp.int32),
            pltpu.VMEM((CH,), jnp.float32),
            pltpu.VMEM((CH,), jnp.float32),
            pltpu.VMEM((CH,), jnp.float32),
            pltpu.VMEM((CH, C), jnp.float32),
            pltpu.VMEM((CH, C), jnp.float32),
            pltpu.VMEM((CH, C), jnp.float32),
            pltpu.VMEM((BND_PAD,), jnp.int32),
            pltpu.SemaphoreType.DMA,
            pltpu.SemaphoreType.DMA,
            pltpu.SemaphoreType.DMA,
            pltpu.SemaphoreType.DMA,
            pltpu.SemaphoreType.DMA,
            pltpu.SemaphoreType.DMA,
        ],
        compiler_params=pltpu.CompilerParams(use_tc_tiling_on_sc=False,
                                             needs_layout_passes=False),
    )
    return f(depth_flat, feat2, idx3, bnd)


def kernel(depth, feat, ranks_depths, ranks_feats, ranks_bevs, bev_feat_shape,
           interval_starts, interval_lengths):
    B = depth.shape[0]
    Cc = feat.shape[-1]
    Z, Yb, Xb = 1, 256, 256
    Bt, Zt, Yt, Xt, Ct = bev_feat_shape
    shape_residual = (Bt + Zt + Yt + Xt + Ct) - (B + Z + Yb + Xb + Cc)

    depth_flat = depth.reshape(-1)
    feat2 = feat.reshape(-1, Cc)

    sent = jnp.full((P_PAD,), NBEV, dtype=jnp.int32)
    zpad = jnp.zeros((P_PAD,), dtype=jnp.int32)
    idx3 = jnp.concatenate(
        [ranks_bevs, sent, ranks_depths, zpad, ranks_feats, zpad]
    ).reshape(3, -1)

    boundaries = jnp.arange(0, NBEV + 1, RROWS, dtype=jnp.int32)
    bnd = jnp.searchsorted(ranks_bevs, boundaries).astype(jnp.int32)
    bnd = jnp.concatenate([
        bnd,
        jnp.asarray(shape_residual, dtype=jnp.int32).reshape(1),
        jnp.zeros((BND_PAD - NR - 2,), dtype=jnp.int32),
    ])

    out_flat = _bev_pool(depth_flat, feat2, idx3, bnd)
    out = out_flat.reshape(B, Z, Yb, Xb, Cc)
    return jnp.transpose(out, (0, 4, 1, 2, 3))


# speedup vs baseline: 1.0511x; 1.0511x over previous
import functools

import jax
import jax.numpy as jnp
from jax import lax
from jax.experimental import pallas as pl
from jax.experimental.pallas import tpu as pltpu
from jax.experimental.pallas import tpu_sc as plsc

NC = 2
NS = 16
NW = NC * NS
LANES = 16

NBEV = 65536
C = 80
NR = 128
RROWS = NBEV // NR
RPW = NR // NW
CH = 256
IDXC = 128
NBUF = 3
P_PAD = CH
BND_PAD = 152

_GDN = lax.GatherDimensionNumbers(
    offset_dims=(), collapsed_slice_dims=(0,), start_index_map=(0,))


def _bev_body(depth_hbm, feat_hbm, idx_hbm, bnd_hbm, out_hbm,
              acc, stg0, stg1, stg2, dv0, dv1, dv2, fv0, fv1, fv2, bndv,
              sems0, sems1, sems2, semg0a, semg0b, semg1a, semg1b,
              semg2a, semg2b):
    cid = lax.axis_index("c")
    sid = lax.axis_index("s")
    wid = sid * NC + cid
    stg = (stg0, stg1, stg2)
    dv = (dv0, dv1, dv2)
    fv = (fv0, fv1, fv2)
    sem_s = (sems0, sems1, sems2)
    sem_g = ((semg0a, semg0b), (semg1a, semg1b), (semg2a, semg2b))

    pltpu.sync_copy(bnd_hbm, bndv)
    resid = bndv[pl.ds(NR + 1, LANES)][0].astype(jnp.float32)
    rsplat = jnp.full((LANES,), 1.0, dtype=jnp.float32) * resid

    for r in range(RPW):
        rng = wid * RPW + r
        base = rng * RROWS

        def zbody(i, carry):
            acc[pl.ds(i * LANES, LANES)] = rsplat
            return carry
        lax.fori_loop(0, RROWS * C // LANES, zbody, 0)

        lo = bndv[pl.ds(rng, LANES)][0]
        hi = bndv[pl.ds(rng + 1, LANES)][0]
        lo_a = (lo // 8) * 8
        n = hi - lo_a
        nch = (n + CH - 1) // CH

        def fire_stage(j, k):
            offs = lo_a + j * CH
            pltpu.async_copy(idx_hbm.at[:, pl.ds(offs, CH)], stg[k], sem_s[k])

        def wait_stage(j, k):
            pltpu.make_async_copy(idx_hbm.at[:, pl.ds(lo_a + j * CH, CH)],
                                  stg[k], sem_s[k]).wait()

        def fire_gathers(k):
            for h in range(CH // IDXC):
                sl = pl.ds(h * IDXC, IDXC)
                pltpu.async_copy(depth_hbm.at[stg[k].at[1, sl]],
                                 dv[k].at[sl], sem_g[k][h])
                pltpu.async_copy(feat_hbm.at[stg[k].at[2, sl]],
                                 fv[k].at[sl, :], sem_g[k][h])

        def wait_gathers(k, h):
            sl = pl.ds(h * IDXC, IDXC)
            pltpu.make_async_copy(depth_hbm.at[stg[k].at[1, sl]],
                                  dv[k].at[sl], sem_g[k][h]).wait()
            pltpu.make_async_copy(feat_hbm.at[stg[k].at[2, sl]],
                                  fv[k].at[sl, :], sem_g[k][h]).wait()

        def compute(k, h):
            iota = lax.iota(jnp.int32, LANES)

            def pt_body(p):
                gp = (p // LANES) * LANES
                lane = p - gp
                rb16 = stg[k][0, pl.ds(gp, LANES)]
                d16 = dv[k][pl.ds(gp, LANES)]
                lanev = jnp.full((LANES, 1), lane, dtype=jnp.int32)
                bevb = lax.gather(rb16, lanev, _GDN, (1,),
                                  mode=lax.GatherScatterMode.PROMISE_IN_BOUNDS)
                db = lax.gather(d16, lanev, _GDN, (1,),
                                mode=lax.GatherScatterMode.PROMISE_IN_BOUNDS)
                okv = jnp.logical_and(bevb >= base, bevb < base + RROWS)
                idx0 = (bevb - base) * C + iota
                for cg in range(C // LANES):
                    x = fv[k][p, pl.ds(cg * LANES, LANES)]
                    plsc.addupdate_scatter(
                        acc, [idx0 + (cg * LANES) if cg else idx0],
                        x * db, mask=okv)
            plsc.parallel_loop(h * IDXC, (h + 1) * IDXC, 1, unroll=8)(pt_body)

        @pl.when(nch > 0)
        def _():
            fire_stage(0, 0)
            wait_stage(0, 0)
            fire_gathers(0)

        @pl.when(nch > 1)
        def _():
            fire_stage(1, 1)

        def ring_body(jj, carry):
            for b in range(NBUF):
                j = jj * NBUF + b

                @pl.when(j < nch)
                def _():
                    @pl.when(j + 2 < nch)
                    def _():
                        fire_stage(j + 2, (b + 2) % NBUF)

                    @pl.when(j + 1 < nch)
                    def _():
                        wait_stage(j + 1, (b + 1) % NBUF)
                        fire_gathers((b + 1) % NBUF)

                    for h in range(CH // IDXC):
                        wait_gathers(b, h)
                        compute(b, h)
            return carry
        lax.fori_loop(0, (nch + NBUF - 1) // NBUF, ring_body, 0)

        pltpu.sync_copy(acc, out_hbm.at[pl.ds(base * C, RROWS * C)])


@functools.partial(jax.jit, donate_argnums=())
def _bev_pool(depth_flat, feat2, idx3, bnd):
    mesh = plsc.VectorSubcoreMesh(core_axis_name="c", subcore_axis_name="s",
                                  num_cores=NC, num_subcores=NS)
    f = pl.kernel(
        _bev_body,
        out_type=jax.ShapeDtypeStruct((NBEV * C,), jnp.float32),
        mesh=mesh,
        scratch_types=[
            pltpu.VMEM((RROWS * C,), jnp.float32),
            pltpu.VMEM((3, CH), jnp.int32),
            pltpu.VMEM((3, CH), jnp.int32),
            pltpu.VMEM((3, CH), jnp.int32),
            pltpu.VMEM((CH,), jnp.float32),
            pltpu.VMEM((CH,), jnp.float32),
            pltpu.VMEM((CH,), jnp.float32),
            pltpu.VMEM((CH, C), jnp.float32),
            pltpu.VMEM((CH, C), jnp.float32),
            pltpu.VMEM((CH, C), jnp.float32),
            pltpu.VMEM((BND_PAD,), jnp.int32),
            pltpu.SemaphoreType.DMA,
            pltpu.SemaphoreType.DMA,
            pltpu.SemaphoreType.DMA,
            pltpu.SemaphoreType.DMA,
            pltpu.SemaphoreType.DMA,
            pltpu.SemaphoreType.DMA,
            pltpu.SemaphoreType.DMA,
            pltpu.SemaphoreType.DMA,
            pltpu.SemaphoreType.DMA,
        ],
        compiler_params=pltpu.CompilerParams(use_tc_tiling_on_sc=False,
                                             needs_layout_passes=False),
    )
    return f(depth_flat, feat2, idx3, bnd)


def kernel(depth, feat, ranks_depths, ranks_feats, ranks_bevs, bev_feat_shape,
           interval_starts, interval_lengths):
    B = depth.shape[0]
    Cc = feat.shape[-1]
    Z, Yb, Xb = 1, 256, 256
    Bt, Zt, Yt, Xt, Ct = bev_feat_shape
    shape_residual = (Bt + Zt + Yt + Xt + Ct) - (B + Z + Yb + Xb + Cc)

    depth_flat = depth.reshape(-1)
    feat2 = feat.reshape(-1, Cc)

    sent = jnp.full((P_PAD,), NBEV, dtype=jnp.int32)
    zpad = jnp.zeros((P_PAD,), dtype=jnp.int32)
    idx3 = jnp.concatenate(
        [ranks_bevs, sent, ranks_depths, zpad, ranks_feats, zpad]
    ).reshape(3, -1)

    boundaries = jnp.arange(0, NBEV + 1, RROWS, dtype=jnp.int32)
    bnd = jnp.searchsorted(ranks_bevs, boundaries).astype(jnp.int32)
    bnd = jnp.concatenate([
        bnd,
        jnp.asarray(shape_residual, dtype=jnp.int32).reshape(1),
        jnp.zeros((BND_PAD - NR - 2,), dtype=jnp.int32),
    ])

    out_flat = _bev_pool(depth_flat, feat2, idx3, bnd)
    out = out_flat.reshape(B, Z, Yb, Xb, Cc)
    return jnp.transpose(out, (0, 4, 1, 2, 3))


# speedup vs baseline: 1.0711x; 1.0190x over previous
import functools

import jax
import jax.numpy as jnp
from jax import lax
from jax.experimental import pallas as pl
from jax.experimental.pallas import tpu as pltpu
from jax.experimental.pallas import tpu_sc as plsc

NC = 2
NS = 16
NW = NC * NS
LANES = 16

NBEV = 65536
C = 80
NR = 128
RROWS = NBEV // NR
RPW = NR // NW
CH = 256
IDXC = 128
NBUF = 3
P_PAD = CH
BND_PAD = 152

_GDN = lax.GatherDimensionNumbers(
    offset_dims=(), collapsed_slice_dims=(0,), start_index_map=(0,))


def _bev_body(depth_hbm, feat_hbm, idx_hbm, bnd_hbm, out_hbm,
              acc, stg0, stg1, stg2, dv0, dv1, dv2, fv0, fv1, fv2,
              bndv, sems0, sems1, sems2, semg0, semg1, semg2):
    cid = lax.axis_index("c")
    sid = lax.axis_index("s")
    wid = sid * NC + cid
    stg = (stg0, stg1, stg2)
    dv = (dv0, dv1, dv2)
    fv = (fv0, fv1, fv2)
    sem_s = (sems0, sems1, sems2)
    sem_g = (semg0, semg1, semg2)

    pltpu.sync_copy(bnd_hbm, bndv)
    resid = bndv[pl.ds(NR + 1, LANES)][0].astype(jnp.float32)
    rsplat = jnp.full((LANES,), 1.0, dtype=jnp.float32) * resid

    for r in range(RPW):
        rng = wid * RPW + r
        base = rng * RROWS

        def zbody(i, carry):
            acc[pl.ds(i * LANES, LANES)] = rsplat
            return carry
        lax.fori_loop(0, RROWS * C // LANES, zbody, 0)

        lo = bndv[pl.ds(rng, LANES)][0]
        hi = bndv[pl.ds(rng + 1, LANES)][0]
        lo_a = (lo // 8) * 8
        n = hi - lo_a
        nch = (n + CH - 1) // CH

        def fire_stage(j, k):
            offs = lo_a + j * CH
            pltpu.async_copy(idx_hbm.at[:, pl.ds(offs, CH)], stg[k], sem_s[k])

        def wait_stage(j, k):
            pltpu.make_async_copy(idx_hbm.at[:, pl.ds(lo_a + j * CH, CH)],
                                  stg[k], sem_s[k]).wait()

        def fire_gathers(k):
            for h in range(CH // IDXC):
                sl = pl.ds(h * IDXC, IDXC)
                pltpu.async_copy(depth_hbm.at[stg[k].at[1, sl]],
                                 dv[k].at[sl], sem_g[k])
                pltpu.async_copy(feat_hbm.at[stg[k].at[2, sl]],
                                 fv[k].at[sl, :], sem_g[k])

        def wait_gathers(k):
            for h in range(CH // IDXC):
                sl = pl.ds(h * IDXC, IDXC)
                pltpu.make_async_copy(depth_hbm.at[stg[k].at[1, sl]],
                                      dv[k].at[sl], sem_g[k]).wait()
                pltpu.make_async_copy(feat_hbm.at[stg[k].at[2, sl]],
                                      fv[k].at[sl, :], sem_g[k]).wait()

        def compute(k):
            iota = lax.iota(jnp.int32, LANES)

            def pt_body(p):
                gp = (p // LANES) * LANES
                lane = p - gp
                rb16 = stg[k][0, pl.ds(gp, LANES)]
                d16 = dv[k][pl.ds(gp, LANES)]
                lanev = jnp.full((LANES, 1), lane, dtype=jnp.int32)
                bevb = lax.gather(rb16, lanev, _GDN, (1,),
                                  mode=lax.GatherScatterMode.PROMISE_IN_BOUNDS)
                db = lax.gather(d16, lanev, _GDN, (1,),
                                mode=lax.GatherScatterMode.PROMISE_IN_BOUNDS)
                okv = jnp.logical_and(bevb >= base, bevb < base + RROWS)
                idx0 = (bevb - base) * C + iota
                for cg in range(C // LANES):
                    x = fv[k][p, pl.ds(cg * LANES, LANES)]
                    plsc.addupdate_scatter(
                        acc, [idx0 + (cg * LANES) if cg else idx0],
                        x * db, mask=okv)
            plsc.parallel_loop(0, CH, 1, unroll=8)(pt_body)

        @pl.when(nch > 0)
        def _():
            fire_stage(0, 0)
            wait_stage(0, 0)
            fire_gathers(0)

        @pl.when(nch > 1)
        def _():
            fire_stage(1, 1)

        def ring_body(jj, carry):
            for b in range(NBUF):
                j = jj * NBUF + b

                @pl.when(j < nch)
                def _():
                    @pl.when(j + 2 < nch)
                    def _():
                        fire_stage(j + 2, (b + 2) % NBUF)

                    @pl.when(j + 1 < nch)
                    def _():
                        wait_stage(j + 1, (b + 1) % NBUF)
                        fire_gathers((b + 1) % NBUF)

                    wait_gathers(b)
                    compute(b)
            return carry
        lax.fori_loop(0, (nch + NBUF - 1) // NBUF, ring_body, 0)

        pltpu.sync_copy(acc, out_hbm.at[pl.ds(base * C, RROWS * C)])


@functools.partial(jax.jit, donate_argnums=())
def _bev_pool(depth_flat, feat2, idx3, bnd):
    mesh = plsc.VectorSubcoreMesh(core_axis_name="c", subcore_axis_name="s",
                                  num_cores=NC, num_subcores=NS)
    f = pl.kernel(
        _bev_body,
        out_type=jax.ShapeDtypeStruct((NBEV * C,), jnp.float32),
        mesh=mesh,
        scratch_types=[
            pltpu.VMEM((RROWS * C,), jnp.float32),
            pltpu.VMEM((3, CH), jnp.int32),
            pltpu.VMEM((3, CH), jnp.int32),
            pltpu.VMEM((3, CH), jnp.int32),
            pltpu.VMEM((CH,), jnp.float32),
            pltpu.VMEM((CH,), jnp.float32),
            pltpu.VMEM((CH,), jnp.float32),
            pltpu.VMEM((CH, C), jnp.float32),
            pltpu.VMEM((CH, C), jnp.float32),
            pltpu.VMEM((CH, C), jnp.float32),
            pltpu.VMEM((BND_PAD,), jnp.int32),
            pltpu.SemaphoreType.DMA,
            pltpu.SemaphoreType.DMA,
            pltpu.SemaphoreType.DMA,
            pltpu.SemaphoreType.DMA,
            pltpu.SemaphoreType.DMA,
            pltpu.SemaphoreType.DMA,
        ],
        compiler_params=pltpu.CompilerParams(use_tc_tiling_on_sc=False,
                                             needs_layout_passes=False),
    )
    return f(depth_flat, feat2, idx3, bnd)


def kernel(depth, feat, ranks_depths, ranks_feats, ranks_bevs, bev_feat_shape,
           interval_starts, interval_lengths):
    B = depth.shape[0]
    Cc = feat.shape[-1]
    Z, Yb, Xb = 1, 256, 256
    Bt, Zt, Yt, Xt, Ct = bev_feat_shape
    shape_residual = (Bt + Zt + Yt + Xt + Ct) - (B + Z + Yb + Xb + Cc)

    depth_flat = depth.reshape(-1)
    feat2 = feat.reshape(-1, Cc)

    sent = jnp.full((P_PAD,), NBEV, dtype=jnp.int32)
    zpad = jnp.zeros((P_PAD,), dtype=jnp.int32)
    idx3 = jnp.concatenate(
        [ranks_bevs, sent, ranks_depths, zpad, ranks_feats, zpad]
    ).reshape(3, -1)

    boundaries = jnp.arange(0, NBEV + 1, RROWS, dtype=jnp.int32)
    bnd = jnp.searchsorted(ranks_bevs, boundaries).astype(jnp.int32)
    bnd = jnp.concatenate([
        bnd,
        jnp.asarray(shape_residual, dtype=jnp.int32).reshape(1),
        jnp.zeros((BND_PAD - NR - 2,), dtype=jnp.int32),
    ])

    out_flat = _bev_pool(depth_flat, feat2, idx3, bnd)
    out = out_flat.reshape(B, Z, Yb, Xb, Cc)
    return jnp.transpose(out, (0, 4, 1, 2, 3))
